# baseline (speedup 1.0000x reference)
import jax
import jax.numpy as jnp
from jax import lax
from jax.experimental import pallas as pl
from jax.experimental.pallas import tpu as pltpu

_RB = 256


def _gcn_body(a_ref, x_ref, w_ref, b_ref, o_ref, tri_ref, d_ref, ybf_ref, acc_ref):
    k = pl.program_id(0)
    ns = d_ref.shape[0]
    off = [_RB * c * (c - 1) // 2 for c in range(ns + 1)]

    for c in range(ns):
        @pl.when(k == c + 1)
        def _stripe(c=c):
            a = a_ref[...]
            deg = jnp.sum(a, axis=1, keepdims=True) + 1.0
            d = lax.rsqrt(deg)
            d_ref[pl.ds(c, 1)] = d[None]
            xw = lax.dot_general(
                x_ref[...], w_ref[...],
                dimension_numbers=(((1,), (1,)), ((), ())),
                preferred_element_type=jnp.float32,
            )
            yc = (d * xw).astype(jnp.bfloat16)
            ybf_ref[pl.ds(c * _RB, _RB), :] = yc

            last = c == ns - 1

            lo = a[:, 0:(c + 1) * _RB].astype(jnp.bfloat16)
            z1 = lax.dot_general(
                lo, ybf_ref[0:(c + 1) * _RB, :],
                dimension_numbers=(((1,), (0,)), ((), ())),
                preferred_element_type=jnp.float32,
            )
            if last:
                o_ref[pl.ds(c * _RB, _RB), :] = (
                    d * z1 + d * yc.astype(jnp.float32) + b_ref[...])
            else:
                acc_ref[pl.ds(c * _RB, _RB), :] = z1

            for j in range(c + 1, ns):
                tri_ref[off[j] + c * _RB:off[j] + (c + 1) * _RB, :] = (
                    a[:, j * _RB:(j + 1) * _RB].astype(jnp.bfloat16))

            if c > 0:
                rows = c * _RB
                z2 = lax.dot_general(
                    tri_ref[off[c]:off[c] + rows, :], yc,
                    dimension_numbers=(((1,), (0,)), ((), ())),
                    preferred_element_type=jnp.float32,
                )
                if last:
                    for i in range(c):
                        di = d_ref[pl.ds(i, 1)][0]
                        yi = ybf_ref[pl.ds(i * _RB, _RB), :].astype(jnp.float32)
                        zi = z2[i * _RB:(i + 1) * _RB, :]
                        ai = acc_ref[pl.ds(i * _RB, _RB), :]
                        o_ref[pl.ds(i * _RB, _RB), :] = (
                            di * (ai + zi) + di * yi + b_ref[...])
                else:
                    acc_ref[0:rows, :] += z2


def kernel(x, A, W, b):
    n, din = x.shape
    dout = W.shape[0]
    ns = n // _RB
    tri_rows = _RB * ns * (ns - 1) // 2

    out = pl.pallas_call(
        _gcn_body,
        grid=(ns + 1,),
        in_specs=[
            pl.BlockSpec((_RB, n), lambda k: (jnp.clip(k - 1, 0, ns - 1), 0)),
            pl.BlockSpec((_RB, din), lambda k: (jnp.clip(k - 1, 0, ns - 1), 0)),
            pl.BlockSpec((dout, din), lambda k: (0, 0)),
            pl.BlockSpec((1, dout), lambda k: (0, 0)),
        ],
        out_specs=pl.BlockSpec((n, dout), lambda k: (0, 0)),
        out_shape=jax.ShapeDtypeStruct((n, dout), jnp.float32),
        scratch_shapes=[
            pltpu.VMEM((tri_rows, _RB), jnp.bfloat16),
            pltpu.VMEM((ns, _RB, 1), jnp.float32),
            pltpu.VMEM((n, dout), jnp.bfloat16),
            pltpu.VMEM((n, dout), jnp.float32),
        ],
    )(A, x, W, b.reshape(1, dout))
    return out


# speedup vs baseline: 1.0300x; 1.0300x over previous
import jax
import jax.numpy as jnp
from jax import lax
from jax.experimental import pallas as pl
from jax.experimental.pallas import tpu as pltpu

_RB = 512
_TB = 256
_NS1 = 7


def _plan(n):
    chunks = [(i * _RB, _RB) for i in range(_NS1)]
    base = _NS1 * _RB
    while base < n:
        chunks.append((base, _TB))
        base += _TB
    return chunks


def _gcn_body(a1_ref, a2_ref, x_ref, w_ref, b_ref, o_ref,
              t512_ref, t256_ref, d_ref, ybf_ref, acc_ref):
    k = pl.program_id(0)
    n = x_ref.shape[0]
    chunks = _plan(n)
    nch = len(chunks)
    off = {}
    pos = {_RB: 0, _TB: 0}
    for (s, h) in chunks:
        off[s] = pos[h]
        pos[h] += s

    for ci, (s, h) in enumerate(chunks):
        @pl.when(k == ci + 1)
        def _chunk(ci=ci, s=s, h=h):
            a = a1_ref[...] if h == _RB else a2_ref[...]
            last = ci == nch - 1

            deg = jnp.sum(a, axis=1, keepdims=True) + 1.0
            d = lax.rsqrt(deg)
            d_ref[pl.ds(s, h), :] = d
            xw = lax.dot_general(
                x_ref[s:s + h, :], w_ref[...],
                dimension_numbers=(((1,), (1,)), ((), ())),
                preferred_element_type=jnp.float32,
            )
            yc = (d * xw).astype(jnp.bfloat16)
            ybf_ref[pl.ds(s, h), :] = yc

            lo = a[:, 0:s + h].astype(jnp.bfloat16)
            z1 = lax.dot_general(
                lo, ybf_ref[0:s + h, :],
                dimension_numbers=(((1,), (0,)), ((), ())),
                preferred_element_type=jnp.float32,
            )
            if last:
                o_ref[pl.ds(s, h), :] = (
                    d * z1 + d * yc.astype(jnp.float32) + b_ref[...])
            else:
                acc_ref[pl.ds(s, h), :] = z1

            for (sj, hj) in chunks[ci + 1:]:
                tri = t512_ref if hj == _RB else t256_ref
                tri[off[sj] + s:off[sj] + s + h, :] = (
                    a[:, sj:sj + hj].astype(jnp.bfloat16))

            if ci > 0:
                tri = t512_ref if h == _RB else t256_ref
                z2 = lax.dot_general(
                    tri[off[s]:off[s] + s, :], yc,
                    dimension_numbers=(((1,), (0,)), ((), ())),
                    preferred_element_type=jnp.float32,
                )
                if last:
                    for (si, hi) in chunks[:ci]:
                        di = d_ref[pl.ds(si, hi), :]
                        yi = ybf_ref[pl.ds(si, hi), :].astype(jnp.float32)
                        zi = z2[si:si + hi, :]
                        ai = acc_ref[pl.ds(si, hi), :]
                        o_ref[pl.ds(si, hi), :] = (
                            di * (ai + zi) + di * yi + b_ref[...])
                else:
                    acc_ref[0:s, :] += z2


def kernel(x, A, W, b):
    n, din = x.shape
    dout = W.shape[0]
    chunks = _plan(n)
    nch = len(chunks)
    ntail = nch - _NS1
    t512_rows = sum(s for (s, h) in chunks if h == _RB)
    t256_rows = sum(s for (s, h) in chunks if h == _TB)
    first_tail = _NS1 * _RB // _TB

    def a2_map(k):
        t = jnp.clip(k - (_NS1 + 1), 0, ntail - 1)
        return (first_tail + t, 0)

    out = pl.pallas_call(
        _gcn_body,
        grid=(nch + 1,),
        in_specs=[
            pl.BlockSpec((_RB, n), lambda k: (jnp.clip(k - 1, 0, _NS1 - 1), 0)),
            pl.BlockSpec((_TB, n), a2_map),
            pl.BlockSpec((n, din), lambda k: (0, 0)),
            pl.BlockSpec((dout, din), lambda k: (0, 0)),
            pl.BlockSpec((1, dout), lambda k: (0, 0)),
        ],
        out_specs=pl.BlockSpec((n, dout), lambda k: (0, 0)),
        out_shape=jax.ShapeDtypeStruct((n, dout), jnp.float32),
        scratch_shapes=[
            pltpu.VMEM((t512_rows, _RB), jnp.bfloat16),
            pltpu.VMEM((t256_rows, _TB), jnp.bfloat16),
            pltpu.VMEM((n, 1), jnp.float32),
            pltpu.VMEM((n, dout), jnp.bfloat16),
            pltpu.VMEM((n, dout), jnp.float32),
        ],
    )(A, A, x, W, b.reshape(1, dout))
    return out


# speedup vs baseline: 1.1071x; 1.0748x over previous
import jax
import jax.numpy as jnp
from jax import lax
from jax.experimental import pallas as pl
from jax.experimental.pallas import tpu as pltpu

_RB = 512


def _gcn_body(a_ref, x_ref, w_ref, b_ref, o_ref, tri_ref, d_ref, ybf_ref, acc_ref):
    k = pl.program_id(0)
    ns = d_ref.shape[0]
    off = [_RB * c * (c - 1) // 2 for c in range(ns + 1)]

    for c in range(ns):
        @pl.when(k == c + 1)
        def _stripe(c=c):
            a = a_ref[...]
            deg = jnp.sum(a, axis=1, keepdims=True) + 1.0
            d = lax.rsqrt(deg)
            d_ref[pl.ds(c, 1)] = d[None]
            xw = lax.dot_general(
                x_ref[...], w_ref[...],
                dimension_numbers=(((1,), (1,)), ((), ())),
                preferred_element_type=jnp.float32,
            )
            yc = (d * xw).astype(jnp.bfloat16)
            ybf_ref[pl.ds(c * _RB, _RB), :] = yc

            last = c == ns - 1

            lo = a[:, 0:(c + 1) * _RB].astype(jnp.bfloat16)
            z1 = lax.dot_general(
                lo, ybf_ref[0:(c + 1) * _RB, :],
                dimension_numbers=(((1,), (0,)), ((), ())),
                preferred_element_type=jnp.float32,
            )
            if last:
                o_ref[pl.ds(c * _RB, _RB), :] = (
                    d * z1 + d * yc.astype(jnp.float32) + b_ref[...])
            else:
                acc_ref[pl.ds(c * _RB, _RB), :] = z1

            for j in range(c + 1, ns):
                tri_ref[off[j] + c * _RB:off[j] + (c + 1) * _RB, :] = (
                    a[:, j * _RB:(j + 1) * _RB].astype(jnp.bfloat16))

            if c > 0:
                rows = c * _RB
                z2 = lax.dot_general(
                    tri_ref[off[c]:off[c] + rows, :], yc,
                    dimension_numbers=(((1,), (0,)), ((), ())),
                    preferred_element_type=jnp.float32,
                )
                if last:
                    for i in range(c):
                        di = d_ref[pl.ds(i, 1)][0]
                        yi = ybf_ref[pl.ds(i * _RB, _RB), :].astype(jnp.float32)
                        zi = z2[i * _RB:(i + 1) * _RB, :]
                        ai = acc_ref[pl.ds(i * _RB, _RB), :]
                        o_ref[pl.ds(i * _RB, _RB), :] = (
                            di * (ai + zi) + di * yi + b_ref[...])
                else:
                    acc_ref[0:rows, :] += z2


def kernel(x, A, W, b):
    n, din = x.shape
    dout = W.shape[0]
    ns = n // _RB
    tri_rows = _RB * ns * (ns - 1) // 2

    out = pl.pallas_call(
        _gcn_body,
        grid=(ns + 1,),
        in_specs=[
            pl.BlockSpec((_RB, n), lambda k: (jnp.clip(k - 1, 0, ns - 1), 0)),
            pl.BlockSpec((_RB, din), lambda k: (jnp.clip(k - 1, 0, ns - 1), 0)),
            pl.BlockSpec((dout, din), lambda k: (0, 0)),
            pl.BlockSpec((1, dout), lambda k: (0, 0)),
        ],
        out_specs=pl.BlockSpec((n, dout), lambda k: (0, 0)),
        out_shape=jax.ShapeDtypeStruct((n, dout), jnp.float32),
        scratch_shapes=[
            pltpu.VMEM((tri_rows, _RB), jnp.bfloat16),
            pltpu.VMEM((ns, _RB, 1), jnp.float32),
            pltpu.VMEM((n, dout), jnp.bfloat16),
            pltpu.VMEM((n, dout), jnp.float32),
        ],
    )(A, x, W, b.reshape(1, dout))
    return out
